# baseline (speedup 1.0000x reference)
import functools

import jax
import jax.numpy as jnp
from jax import lax
from jax.experimental import pallas as pl
from jax.experimental.pallas import tpu as pltpu
from jax.experimental.pallas import tpu_sc as plsc

NC = 2
NS = 16
L = 16
NW = NC * NS
C = 128
NBUF = 2


def _sc_mesh():
    return plsc.VectorSubcoreMesh(core_axis_name="c", subcore_axis_name="s")


@functools.lru_cache(maxsize=None)
def _make_deg(n_pad, n_chunks):
    kpw = n_chunks // NW
    rpt = n_pad // NS

    @functools.partial(
        pl.kernel,
        out_type=jax.ShapeDtypeStruct((NC, n_pad, 16), jnp.float32),
        mesh=_sc_mesh(),
        scratch_types=[
            pltpu.VMEM((kpw, C), jnp.int32),
            pltpu.VMEM((C, 16), jnp.float32),
            pltpu.VMEM((rpt, 16), jnp.float32),
            pltpu.VMEM_SHARED((n_pad, 16), jnp.float32),
        ],
        compiler_params=pltpu.CompilerParams(use_tc_tiling_on_sc=False),
    )
    def deg_kernel(eidx_hbm, out_hbm, didx_v, ones_v, zeros_v, acc_sh):
        c = lax.axis_index("c")
        s = lax.axis_index("s")
        wid = c * NS + s

        def fill_body(i, _):
            ones_v[i] = jnp.ones((L,), jnp.float32)
            return 0

        lax.fori_loop(0, C, fill_body, 0)

        def zero_body(i, _):
            zeros_v[i] = jnp.zeros((L,), jnp.float32)
            return 0

        lax.fori_loop(0, rpt, zero_body, 0)
        pltpu.sync_copy(zeros_v, acc_sh.at[pl.ds(s * rpt, rpt)])
        plsc.subcore_barrier()

        pltpu.sync_copy(eidx_hbm.at[1, pl.ds(wid * kpw, kpw)], didx_v)

        def chunk_body(j, _):
            pltpu.sync_copy(ones_v, acc_sh.at[didx_v.at[j]], add=True)
            return 0

        lax.fori_loop(0, kpw, chunk_body, 0)
        plsc.subcore_barrier()
        pltpu.sync_copy(acc_sh.at[pl.ds(s * rpt, rpt)],
                        out_hbm.at[c, pl.ds(s * rpt, rpt)])

    return deg_kernel


@functools.lru_cache(maxsize=None)
def _make_spmm(n_pad, n_chunks, dim):
    kpw = n_chunks // NW
    rpt = n_pad // NS
    zr = rpt // 2
    assert rpt % zr == 0 and zr % 8 == 0

    R = 2
    NB = 2 * R
    assert kpw % NB == 0

    @functools.partial(
        pl.kernel,
        out_type=jax.ShapeDtypeStruct((NC, n_pad, dim), jnp.float32),
        mesh=_sc_mesh(),
        scratch_types=[
            pltpu.VMEM((2, kpw, C), jnp.int32),
            pltpu.VMEM((NB, C, dim), jnp.float32),
            pltpu.VMEM((zr, dim), jnp.float32),
            pltpu.VMEM_SHARED((n_pad, dim), jnp.float32),
            [pltpu.SemaphoreType.DMA] * NB,
            [pltpu.SemaphoreType.DMA] * NB,
        ],
        compiler_params=pltpu.CompilerParams(use_tc_tiling_on_sc=False),
    )
    def spmm_kernel(eidx_hbm, hs_hbm, out_hbm, idx_v, rows_v, zeros_v, acc_sh,
                    gsems, ssems):
        c = lax.axis_index("c")
        s = lax.axis_index("s")
        wid = c * NS + s

        def zero_body(i, _):
            for k in range(dim // L):
                zeros_v[i, pl.ds(k * L, L)] = jnp.zeros((L,), jnp.float32)
            return 0

        lax.fori_loop(0, zr, zero_body, 0)
        for r in range(rpt // zr):
            pltpu.sync_copy(zeros_v, acc_sh.at[pl.ds(s * rpt + r * zr, zr)])
        plsc.subcore_barrier()

        pltpu.sync_copy(eidx_hbm.at[:, pl.ds(wid * kpw, kpw)], idx_v)

        def gather(j, b):
            return pltpu.async_copy(hs_hbm.at[idx_v.at[0, j]], rows_v.at[b],
                                    gsems[b])

        def scatter(j, b):
            return pltpu.async_copy(rows_v.at[b], acc_sh.at[idx_v.at[1, j]],
                                    ssems[b], add=True)

        for t in range(R):
            gather(t, t)

        def group_body(g, _):
            for b in range(NB):
                j = g * NB + b
                pltpu.make_async_copy(hs_hbm.at[idx_v.at[0, j]],
                                      rows_v.at[b], gsems[b]).wait()
                scatter(j, b)
                fb = (b + R) % NB

                @pl.when(j >= R)
                def _():
                    pltpu.make_async_copy(
                        rows_v.at[fb], acc_sh.at[idx_v.at[1, j - R]],
                        ssems[fb]).wait()

                @pl.when(j + R < kpw)
                def _():
                    gather(j + R, fb)
            return 0

        lax.fori_loop(0, kpw // NB, group_body, 0)
        for i in range(R):
            j = kpw - R + i
            b = j % NB
            pltpu.make_async_copy(rows_v.at[b], acc_sh.at[idx_v.at[1, j]],
                                  ssems[b]).wait()
        plsc.subcore_barrier()
        for r in range(rpt // zr):
            pltpu.sync_copy(
                acc_sh.at[pl.ds(s * rpt + r * zr, zr)],
                out_hbm.at[c, pl.ds(s * rpt + r * zr, zr)])

    return spmm_kernel


def _tc1_body(n, x_ref, w1_ref, degp_ref, dis_ref, h1_ref, h1s_ref):
    deg = degp_ref[0, :n, 0:1] + degp_ref[1, :n, 0:1] + 1.0
    dis = lax.rsqrt(deg)
    h1 = jnp.dot(x_ref[...], w1_ref[...], preferred_element_type=jnp.float32)
    dis_ref[...] = dis
    h1_ref[...] = h1
    h1s_ref[...] = h1 * dis


def _tc_mid_body(n, p_ref, h_ref, dis_ref, b_ref, g_ref, be_ref, w_ref,
                 hn_ref, hns_ref):
    dis = dis_ref[...]
    agg = (p_ref[0, :n] + p_ref[1, :n]) * dis + h_ref[...] * (dis * dis) + b_ref[...]
    h = jnp.maximum(agg, 0.0)
    mu = jnp.mean(h, axis=-1, keepdims=True)
    var = jnp.mean((h - mu) ** 2, axis=-1, keepdims=True)
    h = (h - mu) * lax.rsqrt(var + 1e-5) * g_ref[...] + be_ref[...]
    hn = jnp.dot(h, w_ref[...], preferred_element_type=jnp.float32)
    hn_ref[...] = hn
    hns_ref[...] = hn * dis


def _tc3_body(n, p_ref, h_ref, dis_ref, b_ref, g_ref, be_ref, w_ref,
              hn_ref, lo_ref, hi_ref):
    dis = dis_ref[...]
    agg = (p_ref[0, :n] + p_ref[1, :n]) * dis + h_ref[...] * (dis * dis) + b_ref[...]
    h = jnp.maximum(agg, 0.0)
    mu = jnp.mean(h, axis=-1, keepdims=True)
    var = jnp.mean((h - mu) ** 2, axis=-1, keepdims=True)
    h = (h - mu) * lax.rsqrt(var + 1e-5) * g_ref[...] + be_ref[...]
    hn = jnp.dot(h, w_ref[...], preferred_element_type=jnp.float32)
    hns = hn * dis
    half = hn.shape[1] // 2
    hn_ref[...] = hn
    lo_ref[...] = hns[:, :half]
    hi_ref[...] = hns[:, half:]


def _tc_final_body(n, plo_ref, phi_ref, h_ref, dis_ref, b_ref, pw1_ref,
                   pb1_ref, pw2_ref, pb2_ref, emb_ref, out_ref):
    dis = dis_ref[...]
    agg = jnp.concatenate([
        plo_ref[0, :n] + plo_ref[1, :n],
        phi_ref[0, :n] + phi_ref[1, :n],
    ], axis=-1)
    emb = agg * dis + h_ref[...] * (dis * dis) + b_ref[...]
    emb_ref[...] = emb
    h = jnp.maximum(emb, 0.0)
    h = jnp.dot(h, pw1_ref[...], preferred_element_type=jnp.float32) + pb1_ref[...]
    h = jnp.dot(h, pw2_ref[...], preferred_element_type=jnp.float32) + pb2_ref[...]
    m = jnp.max(h, axis=-1, keepdims=True)
    lse = jnp.log(jnp.sum(jnp.exp(h - m), axis=-1, keepdims=True)) + m
    out_ref[...] = h - lse


def kernel(x, edge_index, W1, b1, W2, b2, W3, b3, ln1_g, ln1_b, ln2_g, ln2_b,
           pW1, pb1, pW2, pb2):
    n, in_dim = x.shape
    e = edge_index.shape[1]
    f32 = jnp.float32

    em = NW * C * NBUF
    epad = ((e + em - 1) // em) * em
    nm = NS * 16
    n_pad = ((n + 1 + nm - 1) // nm) * nm
    if epad != e:
        fake = jnp.concatenate([
            jnp.zeros((1, epad - e), jnp.int32),
            jnp.full((1, epad - e), n, jnp.int32),
        ], axis=0)
        eidx = jnp.concatenate([edge_index.astype(jnp.int32), fake], axis=1)
    else:
        eidx = edge_index.astype(jnp.int32)
    n_chunks = epad // C
    eidx = eidx.reshape(2, n_chunks, C)

    b1r, g1r, be1r = b1.reshape(1, -1), ln1_g.reshape(1, -1), ln1_b.reshape(1, -1)
    b2r, g2r, be2r = b2.reshape(1, -1), ln2_g.reshape(1, -1), ln2_b.reshape(1, -1)
    b3r = b3.reshape(1, -1)
    pb1r, pb2r = pb1.reshape(1, -1), pb2.reshape(1, -1)

    d1, d2, d3 = W1.shape[1], W2.shape[1], W3.shape[1]

    degp = _make_deg(n_pad, n_chunks)(eidx)

    dis, h1, h1s = pl.pallas_call(
        functools.partial(_tc1_body, n),
        out_shape=[
            jax.ShapeDtypeStruct((n, 1), f32),
            jax.ShapeDtypeStruct((n, d1), f32),
            jax.ShapeDtypeStruct((n, d1), f32),
        ],
    )(x, W1, degp)

    p1 = _make_spmm(n_pad, n_chunks, d1)(eidx, h1s)

    h2, h2s = pl.pallas_call(
        functools.partial(_tc_mid_body, n),
        out_shape=[
            jax.ShapeDtypeStruct((n, d2), f32),
            jax.ShapeDtypeStruct((n, d2), f32),
        ],
    )(p1, h1, dis, b1r, g1r, be1r, W2)

    p2 = _make_spmm(n_pad, n_chunks, d2)(eidx, h2s)

    half = d3 // 2
    h3, h3s_lo, h3s_hi = pl.pallas_call(
        functools.partial(_tc3_body, n),
        out_shape=[
            jax.ShapeDtypeStruct((n, d3), f32),
            jax.ShapeDtypeStruct((n, half), f32),
            jax.ShapeDtypeStruct((n, half), f32),
        ],
    )(p2, h2, dis, b2r, g2r, be2r, W3)

    p3_lo = _make_spmm(n_pad, n_chunks, half)(eidx, h3s_lo)
    p3_hi = _make_spmm(n_pad, n_chunks, half)(eidx, h3s_hi)

    emb, out = pl.pallas_call(
        functools.partial(_tc_final_body, n),
        out_shape=[
            jax.ShapeDtypeStruct((n, d3), f32),
            jax.ShapeDtypeStruct((n, pW2.shape[1]), f32),
        ],
    )(p3_lo, p3_hi, h3, dis, b3r, pW1, pb1r, pW2, pb2r)

    return emb, out


# speedup vs baseline: 1.1093x; 1.1093x over previous
import functools

import jax
import jax.numpy as jnp
from jax import lax
from jax.experimental import pallas as pl
from jax.experimental.pallas import tpu as pltpu
from jax.experimental.pallas import tpu_sc as plsc

NC = 2
NS = 16
L = 16
NW = NC * NS
C = 128
NBUF = 2


def _sc_mesh():
    return plsc.VectorSubcoreMesh(core_axis_name="c", subcore_axis_name="s")


@functools.lru_cache(maxsize=None)
def _make_deg(n_pad, n_chunks):
    kpw = n_chunks // NW
    rpt = n_pad // NS

    @functools.partial(
        pl.kernel,
        out_type=jax.ShapeDtypeStruct((NC, n_pad, 16), jnp.float32),
        mesh=_sc_mesh(),
        scratch_types=[
            pltpu.VMEM((kpw, C), jnp.int32),
            pltpu.VMEM((C, 16), jnp.float32),
            pltpu.VMEM((rpt, 16), jnp.float32),
            pltpu.VMEM_SHARED((n_pad, 16), jnp.float32),
        ],
        compiler_params=pltpu.CompilerParams(use_tc_tiling_on_sc=False),
    )
    def deg_kernel(eidx_hbm, out_hbm, didx_v, ones_v, zeros_v, acc_sh):
        c = lax.axis_index("c")
        s = lax.axis_index("s")
        wid = c * NS + s

        def fill_body(i, _):
            ones_v[i] = jnp.ones((L,), jnp.float32)
            return 0

        lax.fori_loop(0, C, fill_body, 0)

        def zero_body(i, _):
            zeros_v[i] = jnp.zeros((L,), jnp.float32)
            return 0

        lax.fori_loop(0, rpt, zero_body, 0)
        pltpu.sync_copy(zeros_v, acc_sh.at[pl.ds(s * rpt, rpt)])
        plsc.subcore_barrier()

        pltpu.sync_copy(eidx_hbm.at[1, pl.ds(wid * kpw, kpw)], didx_v)

        def chunk_body(j, _):
            pltpu.sync_copy(ones_v, acc_sh.at[didx_v.at[j]], add=True)
            return 0

        lax.fori_loop(0, kpw, chunk_body, 0)
        plsc.subcore_barrier()
        pltpu.sync_copy(acc_sh.at[pl.ds(s * rpt, rpt)],
                        out_hbm.at[c, pl.ds(s * rpt, rpt)])

    return deg_kernel


@functools.lru_cache(maxsize=None)
def _make_spmm(n_pad, n_chunks, dim, spmem_table=False):
    kpw = n_chunks // NW
    rpt = n_pad // NS
    zr = rpt // 2
    assert rpt % zr == 0 and zr % 8 == 0

    R = 2
    NB = 2 * R
    assert kpw % NB == 0

    scratch = [
        pltpu.VMEM((2, kpw, C), jnp.int32),
        pltpu.VMEM((NB, C, dim), jnp.float32),
        pltpu.VMEM((zr, dim), jnp.float32),
        pltpu.VMEM_SHARED((n_pad, dim), jnp.float32),
        [pltpu.SemaphoreType.DMA] * NB,
        [pltpu.SemaphoreType.DMA] * NB,
    ]
    if spmem_table:
        scratch.append(pltpu.VMEM_SHARED((n_pad, dim), jnp.float32))

    @functools.partial(
        pl.kernel,
        out_type=jax.ShapeDtypeStruct((NC, n_pad, dim), jnp.float32),
        mesh=_sc_mesh(),
        scratch_types=scratch,
        compiler_params=pltpu.CompilerParams(use_tc_tiling_on_sc=False),
    )
    def spmm_kernel(eidx_hbm, hs_hbm, out_hbm, idx_v, rows_v, zeros_v, acc_sh,
                    gsems, ssems, *maybe_tab):
        c = lax.axis_index("c")
        s = lax.axis_index("s")
        wid = c * NS + s

        def zero_body(i, _):
            for k in range(dim // L):
                zeros_v[i, pl.ds(k * L, L)] = jnp.zeros((L,), jnp.float32)
            return 0

        lax.fori_loop(0, zr, zero_body, 0)
        for r in range(rpt // zr):
            pltpu.sync_copy(zeros_v, acc_sh.at[pl.ds(s * rpt + r * zr, zr)])
        if spmem_table:
            tab = maybe_tab[0]
            pltpu.sync_copy(hs_hbm.at[pl.ds(s * rpt, rpt)],
                            tab.at[pl.ds(s * rpt, rpt)])
            src_ref = tab
        else:
            src_ref = hs_hbm
        plsc.subcore_barrier()

        pltpu.sync_copy(eidx_hbm.at[:, pl.ds(wid * kpw, kpw)], idx_v)

        def gather(j, b):
            return pltpu.async_copy(src_ref.at[idx_v.at[0, j]], rows_v.at[b],
                                    gsems[b])

        def scatter(j, b):
            return pltpu.async_copy(rows_v.at[b], acc_sh.at[idx_v.at[1, j]],
                                    ssems[b], add=True)

        for t in range(R):
            gather(t, t)

        def group_body(g, _):
            for b in range(NB):
                j = g * NB + b
                pltpu.make_async_copy(src_ref.at[idx_v.at[0, j]],
                                      rows_v.at[b], gsems[b]).wait()
                scatter(j, b)
                fb = (b + R) % NB

                @pl.when(j >= R)
                def _():
                    pltpu.make_async_copy(
                        rows_v.at[fb], acc_sh.at[idx_v.at[1, j - R]],
                        ssems[fb]).wait()

                @pl.when(j + R < kpw)
                def _():
                    gather(j + R, fb)
            return 0

        lax.fori_loop(0, kpw // NB, group_body, 0)
        for i in range(R):
            j = kpw - R + i
            b = j % NB
            pltpu.make_async_copy(rows_v.at[b], acc_sh.at[idx_v.at[1, j]],
                                  ssems[b]).wait()
        plsc.subcore_barrier()
        for r in range(rpt // zr):
            pltpu.sync_copy(
                acc_sh.at[pl.ds(s * rpt + r * zr, zr)],
                out_hbm.at[c, pl.ds(s * rpt + r * zr, zr)])

    return spmm_kernel


def _tc1_body(n, x_ref, w1_ref, degp_ref, dis_ref, h1_ref, h1s_ref):
    deg = degp_ref[0, :n, 0:1] + degp_ref[1, :n, 0:1] + 1.0
    dis = lax.rsqrt(deg)
    h1 = jnp.dot(x_ref[...], w1_ref[...], preferred_element_type=jnp.float32)
    dis_ref[...] = dis
    h1_ref[...] = h1
    h1s_ref[:n] = h1 * dis
    h1s_ref[n:] = jnp.zeros_like(h1s_ref[n:])


def _tc_mid_body(n, p_ref, h_ref, dis_ref, b_ref, g_ref, be_ref, w_ref,
                 hn_ref, hns_ref):
    dis = dis_ref[...]
    agg = (p_ref[0, :n] + p_ref[1, :n]) * dis + h_ref[...] * (dis * dis) + b_ref[...]
    h = jnp.maximum(agg, 0.0)
    mu = jnp.mean(h, axis=-1, keepdims=True)
    var = jnp.mean((h - mu) ** 2, axis=-1, keepdims=True)
    h = (h - mu) * lax.rsqrt(var + 1e-5) * g_ref[...] + be_ref[...]
    hn = jnp.dot(h, w_ref[...], preferred_element_type=jnp.float32)
    hn_ref[...] = hn
    hns_ref[...] = hn * dis


def _tc3_body(n, p_ref, h_ref, dis_ref, b_ref, g_ref, be_ref, w_ref,
              hn_ref, lo_ref, hi_ref):
    dis = dis_ref[...]
    agg = (p_ref[0, :n] + p_ref[1, :n]) * dis + h_ref[...] * (dis * dis) + b_ref[...]
    h = jnp.maximum(agg, 0.0)
    mu = jnp.mean(h, axis=-1, keepdims=True)
    var = jnp.mean((h - mu) ** 2, axis=-1, keepdims=True)
    h = (h - mu) * lax.rsqrt(var + 1e-5) * g_ref[...] + be_ref[...]
    hn = jnp.dot(h, w_ref[...], preferred_element_type=jnp.float32)
    hns = hn * dis
    half = hn.shape[1] // 2
    hn_ref[...] = hn
    lo_ref[...] = hns[:, :half]
    hi_ref[...] = hns[:, half:]


def _tc_final_body(n, plo_ref, phi_ref, h_ref, dis_ref, b_ref, pw1_ref,
                   pb1_ref, pw2_ref, pb2_ref, emb_ref, out_ref):
    dis = dis_ref[...]
    agg = jnp.concatenate([
        plo_ref[0, :n] + plo_ref[1, :n],
        phi_ref[0, :n] + phi_ref[1, :n],
    ], axis=-1)
    emb = agg * dis + h_ref[...] * (dis * dis) + b_ref[...]
    emb_ref[...] = emb
    h = jnp.maximum(emb, 0.0)
    h = jnp.dot(h, pw1_ref[...], preferred_element_type=jnp.float32) + pb1_ref[...]
    h = jnp.dot(h, pw2_ref[...], preferred_element_type=jnp.float32) + pb2_ref[...]
    m = jnp.max(h, axis=-1, keepdims=True)
    lse = jnp.log(jnp.sum(jnp.exp(h - m), axis=-1, keepdims=True)) + m
    out_ref[...] = h - lse


def kernel(x, edge_index, W1, b1, W2, b2, W3, b3, ln1_g, ln1_b, ln2_g, ln2_b,
           pW1, pb1, pW2, pb2):
    n, in_dim = x.shape
    e = edge_index.shape[1]
    f32 = jnp.float32

    em = NW * C * NBUF
    epad = ((e + em - 1) // em) * em
    nm = NS * 16
    n_pad = ((n + 1 + nm - 1) // nm) * nm
    if epad != e:
        fake = jnp.concatenate([
            jnp.zeros((1, epad - e), jnp.int32),
            jnp.full((1, epad - e), n, jnp.int32),
        ], axis=0)
        eidx = jnp.concatenate([edge_index.astype(jnp.int32), fake], axis=1)
    else:
        eidx = edge_index.astype(jnp.int32)
    n_chunks = epad // C
    eidx = eidx.reshape(2, n_chunks, C)

    b1r, g1r, be1r = b1.reshape(1, -1), ln1_g.reshape(1, -1), ln1_b.reshape(1, -1)
    b2r, g2r, be2r = b2.reshape(1, -1), ln2_g.reshape(1, -1), ln2_b.reshape(1, -1)
    b3r = b3.reshape(1, -1)
    pb1r, pb2r = pb1.reshape(1, -1), pb2.reshape(1, -1)

    d1, d2, d3 = W1.shape[1], W2.shape[1], W3.shape[1]

    degp = _make_deg(n_pad, n_chunks)(eidx)

    dis, h1, h1s = pl.pallas_call(
        functools.partial(_tc1_body, n),
        out_shape=[
            jax.ShapeDtypeStruct((n, 1), f32),
            jax.ShapeDtypeStruct((n, d1), f32),
            jax.ShapeDtypeStruct((n_pad, d1), f32),
        ],
    )(x, W1, degp)

    p1 = _make_spmm(n_pad, n_chunks, d1, spmem_table=True)(eidx, h1s)

    h2, h2s = pl.pallas_call(
        functools.partial(_tc_mid_body, n),
        out_shape=[
            jax.ShapeDtypeStruct((n, d2), f32),
            jax.ShapeDtypeStruct((n, d2), f32),
        ],
    )(p1, h1, dis, b1r, g1r, be1r, W2)

    p2 = _make_spmm(n_pad, n_chunks, d2)(eidx, h2s)

    half = d3 // 2
    h3, h3s_lo, h3s_hi = pl.pallas_call(
        functools.partial(_tc3_body, n),
        out_shape=[
            jax.ShapeDtypeStruct((n, d3), f32),
            jax.ShapeDtypeStruct((n, half), f32),
            jax.ShapeDtypeStruct((n, half), f32),
        ],
    )(p2, h2, dis, b2r, g2r, be2r, W3)

    p3_lo = _make_spmm(n_pad, n_chunks, half)(eidx, h3s_lo)
    p3_hi = _make_spmm(n_pad, n_chunks, half)(eidx, h3s_hi)

    emb, out = pl.pallas_call(
        functools.partial(_tc_final_body, n),
        out_shape=[
            jax.ShapeDtypeStruct((n, d3), f32),
            jax.ShapeDtypeStruct((n, pW2.shape[1]), f32),
        ],
    )(p3_lo, p3_hi, h3, dis, b3r, pW1, pb1r, pW2, pb2r)

    return emb, out


# speedup vs baseline: 2.3606x; 2.1279x over previous
import functools

import jax
import jax.numpy as jnp
from jax import lax
from jax.experimental import pallas as pl
from jax.experimental.pallas import tpu as pltpu
from jax.experimental.pallas import tpu_sc as plsc

NC = 2
NS = 16
L = 16
NW = NC * NS
C = 128
NBUF = 2


def _sc_mesh():
    return plsc.VectorSubcoreMesh(core_axis_name="c", subcore_axis_name="s")


@functools.lru_cache(maxsize=None)
def _make_deg(n_pad, n_chunks):
    kpw = n_chunks // NW
    rpt = n_pad // NS

    @functools.partial(
        pl.kernel,
        out_type=jax.ShapeDtypeStruct((NC, n_pad, 16), jnp.float32),
        mesh=_sc_mesh(),
        scratch_types=[
            pltpu.VMEM((kpw, C), jnp.int32),
            pltpu.VMEM((C, 16), jnp.float32),
            pltpu.VMEM((rpt, 16), jnp.float32),
            pltpu.VMEM_SHARED((n_pad, 16), jnp.float32),
        ],
        compiler_params=pltpu.CompilerParams(use_tc_tiling_on_sc=False),
    )
    def deg_kernel(eidx_hbm, out_hbm, didx_v, ones_v, zeros_v, acc_sh):
        c = lax.axis_index("c")
        s = lax.axis_index("s")
        wid = c * NS + s

        def fill_body(i, _):
            ones_v[i] = jnp.ones((L,), jnp.float32)
            return 0

        lax.fori_loop(0, C, fill_body, 0)

        def zero_body(i, _):
            zeros_v[i] = jnp.zeros((L,), jnp.float32)
            return 0

        lax.fori_loop(0, rpt, zero_body, 0)
        pltpu.sync_copy(zeros_v, acc_sh.at[pl.ds(s * rpt, rpt)])
        plsc.subcore_barrier()

        pltpu.sync_copy(eidx_hbm.at[1, pl.ds(wid * kpw, kpw)], didx_v)

        def chunk_body(j, _):
            pltpu.sync_copy(ones_v, acc_sh.at[didx_v.at[j]], add=True)
            return 0

        lax.fori_loop(0, kpw, chunk_body, 0)
        plsc.subcore_barrier()
        pltpu.sync_copy(acc_sh.at[pl.ds(s * rpt, rpt)],
                        out_hbm.at[c, pl.ds(s * rpt, rpt)])

    return deg_kernel


SD = 32


@functools.lru_cache(maxsize=None)
def _make_spmm(n_pad, n_chunks, dim):
    kpw = n_chunks // NW
    rpt = n_pad // NS
    zr = rpt // 2
    assert rpt % zr == 0 and zr % 8 == 0
    nt = dim // SD

    R = 2
    NB = 2 * R
    assert kpw % NB == 0

    @functools.partial(
        pl.kernel,
        out_type=jax.ShapeDtypeStruct((NC, n_pad, dim), jnp.float32),
        mesh=_sc_mesh(),
        scratch_types=[
            pltpu.VMEM((2, kpw, C), jnp.int32),
            pltpu.VMEM((NB, C, SD), jnp.float32),
            pltpu.VMEM((zr, SD), jnp.float32),
            pltpu.VMEM_SHARED((n_pad, SD), jnp.float32),
            pltpu.VMEM_SHARED((n_pad, SD), jnp.float32),
            [pltpu.SemaphoreType.DMA] * NB,
            [pltpu.SemaphoreType.DMA] * NB,
        ],
        compiler_params=pltpu.CompilerParams(use_tc_tiling_on_sc=False),
    )
    def spmm_kernel(eidx_hbm, hs_hbm, out_hbm, idx_v, rows_v, zeros_v, acc_sh,
                    tab_sh, gsems, ssems):
        c = lax.axis_index("c")
        s = lax.axis_index("s")
        wid = c * NS + s

        def zero_body(i, _):
            for k in range(SD // L):
                zeros_v[i, pl.ds(k * L, L)] = jnp.zeros((L,), jnp.float32)
            return 0

        lax.fori_loop(0, zr, zero_body, 0)
        pltpu.sync_copy(eidx_hbm.at[:, pl.ds(wid * kpw, kpw)], idx_v)

        def gather(j, b):
            return pltpu.async_copy(tab_sh.at[idx_v.at[0, j]], rows_v.at[b],
                                    gsems[b])

        def scatter(j, b):
            return pltpu.async_copy(rows_v.at[b], acc_sh.at[idx_v.at[1, j]],
                                    ssems[b], add=True)

        for t in range(nt):
            pltpu.sync_copy(
                hs_hbm.at[pl.ds(s * rpt, rpt), pl.ds(t * SD, SD)],
                tab_sh.at[pl.ds(s * rpt, rpt)])
            for r in range(rpt // zr):
                pltpu.sync_copy(zeros_v, acc_sh.at[pl.ds(s * rpt + r * zr, zr)])
            plsc.subcore_barrier()

            for p in range(R):
                gather(p, p)

            def group_body(g, _):
                for b in range(NB):
                    j = g * NB + b
                    pltpu.make_async_copy(tab_sh.at[idx_v.at[0, j]],
                                          rows_v.at[b], gsems[b]).wait()
                    scatter(j, b)
                    fb = (b + R) % NB

                    @pl.when(j >= R)
                    def _():
                        pltpu.make_async_copy(
                            rows_v.at[fb], acc_sh.at[idx_v.at[1, j - R]],
                            ssems[fb]).wait()

                    @pl.when(j + R < kpw)
                    def _():
                        gather(j + R, fb)
                return 0

            lax.fori_loop(0, kpw // NB, group_body, 0)
            for i in range(R):
                j = kpw - R + i
                b = j % NB
                pltpu.make_async_copy(rows_v.at[b], acc_sh.at[idx_v.at[1, j]],
                                      ssems[b]).wait()
            plsc.subcore_barrier()
            for r in range(rpt // zr):
                pltpu.sync_copy(
                    acc_sh.at[pl.ds(s * rpt + r * zr, zr)],
                    out_hbm.at[c, pl.ds(s * rpt + r * zr, zr),
                               pl.ds(t * SD, SD)])

    return spmm_kernel


def _split_tables(hns, n, hns_ref):
    hns_ref[:n] = hns
    hns_ref[n:] = jnp.zeros_like(hns_ref[n:])


def _merge_partials(p_ref, n):
    return p_ref[0, :n] + p_ref[1, :n]


def _tc1_body(n, x_ref, w1_ref, degp_ref, dis_ref, h1_ref, h1s_ref):
    deg = degp_ref[0, :n, 0:1] + degp_ref[1, :n, 0:1] + 1.0
    dis = lax.rsqrt(deg)
    h1 = jnp.dot(x_ref[...], w1_ref[...], preferred_element_type=jnp.float32)
    dis_ref[...] = dis
    h1_ref[...] = h1
    _split_tables(h1 * dis, n, h1s_ref)


def _tc_mid_body(n, p_ref, h_ref, dis_ref, b_ref, g_ref, be_ref, w_ref,
                 hn_ref, hns_ref):
    dis = dis_ref[...]
    agg = _merge_partials(p_ref, n) * dis + h_ref[...] * (dis * dis) + b_ref[...]
    h = jnp.maximum(agg, 0.0)
    mu = jnp.mean(h, axis=-1, keepdims=True)
    var = jnp.mean((h - mu) ** 2, axis=-1, keepdims=True)
    h = (h - mu) * lax.rsqrt(var + 1e-5) * g_ref[...] + be_ref[...]
    hn = jnp.dot(h, w_ref[...], preferred_element_type=jnp.float32)
    hn_ref[...] = hn
    _split_tables(hn * dis, n, hns_ref)


def _tc_final_body(n, p_ref, h_ref, dis_ref, b_ref, pw1_ref,
                   pb1_ref, pw2_ref, pb2_ref, emb_ref, out_ref):
    dis = dis_ref[...]
    emb = _merge_partials(p_ref, n) * dis + h_ref[...] * (dis * dis) + b_ref[...]
    emb_ref[...] = emb
    h = jnp.maximum(emb, 0.0)
    h = jnp.dot(h, pw1_ref[...], preferred_element_type=jnp.float32) + pb1_ref[...]
    h = jnp.dot(h, pw2_ref[...], preferred_element_type=jnp.float32) + pb2_ref[...]
    m = jnp.max(h, axis=-1, keepdims=True)
    lse = jnp.log(jnp.sum(jnp.exp(h - m), axis=-1, keepdims=True)) + m
    out_ref[...] = h - lse


def kernel(x, edge_index, W1, b1, W2, b2, W3, b3, ln1_g, ln1_b, ln2_g, ln2_b,
           pW1, pb1, pW2, pb2):
    n, in_dim = x.shape
    e = edge_index.shape[1]
    f32 = jnp.float32

    em = NW * C * NBUF
    epad = ((e + em - 1) // em) * em
    nm = NS * 16
    n_pad = ((n + 1 + nm - 1) // nm) * nm
    if epad != e:
        fake = jnp.concatenate([
            jnp.zeros((1, epad - e), jnp.int32),
            jnp.full((1, epad - e), n, jnp.int32),
        ], axis=0)
        eidx = jnp.concatenate([edge_index.astype(jnp.int32), fake], axis=1)
    else:
        eidx = edge_index.astype(jnp.int32)
    n_chunks = epad // C
    eidx = eidx.reshape(2, n_chunks, C)

    b1r, g1r, be1r = b1.reshape(1, -1), ln1_g.reshape(1, -1), ln1_b.reshape(1, -1)
    b2r, g2r, be2r = b2.reshape(1, -1), ln2_g.reshape(1, -1), ln2_b.reshape(1, -1)
    b3r = b3.reshape(1, -1)
    pb1r, pb2r = pb1.reshape(1, -1), pb2.reshape(1, -1)

    d1, d2, d3 = W1.shape[1], W2.shape[1], W3.shape[1]

    degp = _make_deg(n_pad, n_chunks)(eidx)

    dis, h1, h1s = pl.pallas_call(
        functools.partial(_tc1_body, n),
        out_shape=[
            jax.ShapeDtypeStruct((n, 1), f32),
            jax.ShapeDtypeStruct((n, d1), f32),
            jax.ShapeDtypeStruct((n_pad, d1), f32),
        ],
    )(x, W1, degp)

    p1 = _make_spmm(n_pad, n_chunks, d1)(eidx, h1s)

    h2, h2s = pl.pallas_call(
        functools.partial(_tc_mid_body, n),
        out_shape=[
            jax.ShapeDtypeStruct((n, d2), f32),
            jax.ShapeDtypeStruct((n_pad, d2), f32),
        ],
    )(p1, h1, dis, b1r, g1r, be1r, W2)

    p2 = _make_spmm(n_pad, n_chunks, d2)(eidx, h2s)

    h3, h3s = pl.pallas_call(
        functools.partial(_tc_mid_body, n),
        out_shape=[
            jax.ShapeDtypeStruct((n, d3), f32),
            jax.ShapeDtypeStruct((n_pad, d3), f32),
        ],
    )(p2, h2, dis, b2r, g2r, be2r, W3)

    p3 = _make_spmm(n_pad, n_chunks, d3)(eidx, h3s)

    emb, out = pl.pallas_call(
        functools.partial(_tc_final_body, n),
        out_shape=[
            jax.ShapeDtypeStruct((n, d3), f32),
            jax.ShapeDtypeStruct((n, pW2.shape[1]), f32),
        ],
    )(p3, h3, dis, b3r, pW1, pb1r, pW2, pb2r)

    return emb, out
